# Initial kernel scaffold; baseline (speedup 1.0000x reference)
import functools

import jax
import jax.numpy as jnp
from jax import lax
from jax.experimental import pallas as pl
from jax.experimental.pallas import tpu as pltpu
from jax.experimental.pallas import tpu_sc as plsc

NUM_ROWS = 16384 * 50
DIM = 32
NC, NS = 2, 16
NW = NC * NS
PER_W = NUM_ROWS // NW
SEG = 128
GROUPS = PER_W // SEG
K = 20
NBATCH = GROUPS // K
ROWS_PER_BATCH = K * SEG


def _sc_gather(idx, table):
    mesh = plsc.VectorSubcoreMesh(core_axis_name="c", subcore_axis_name="s")

    @functools.partial(
        pl.kernel,
        mesh=mesh,
        out_type=jax.ShapeDtypeStruct((NUM_ROWS, DIM), jnp.float32),
        scratch_types=[
            pltpu.VMEM((GROUPS, SEG), jnp.int32),
            pltpu.VMEM((ROWS_PER_BATCH, DIM), jnp.float32),
            pltpu.SemaphoreType.DMA,
        ],
    )
    def k(idx_hbm, table_hbm, out_hbm, idx_v, rows_v, sem):
        wid = lax.axis_index("s") * NC + lax.axis_index("c")
        base = wid * PER_W
        pltpu.sync_copy(idx_hbm.at[wid], idx_v)

        def body(b, _):
            copies = []
            for j in range(K):
                copies.append(
                    pltpu.async_copy(
                        table_hbm.at[idx_v.at[b * K + j]],
                        rows_v.at[pl.ds(j * SEG, SEG)],
                        sem,
                    )
                )
            for c in copies:
                c.wait()
            pltpu.sync_copy(
                rows_v, out_hbm.at[pl.ds(base + b * ROWS_PER_BATCH, ROWS_PER_BATCH)]
            )
            return ()

        lax.fori_loop(0, NBATCH, body, ())

    return k(idx, table)


def kernel(token_ids, embeddings):
    idx = token_ids.reshape(NW, GROUPS, SEG).astype(jnp.int32)
    out = _sc_gather(idx, embeddings)
    return out.reshape(token_ids.shape + (DIM,))


# speedup vs baseline: 1.1107x; 1.1107x over previous
import functools

import jax
import jax.numpy as jnp
from jax import lax
from jax.experimental import pallas as pl
from jax.experimental.pallas import tpu as pltpu
from jax.experimental.pallas import tpu_sc as plsc

NUM_ROWS = 16384 * 50
DIM = 32
NC, NS = 2, 16
NW = NC * NS
PER_W = NUM_ROWS // NW
SEG = 128
GROUPS = PER_W // SEG
K = 20
NBATCH = GROUPS // K
ROWS_PER_BATCH = K * SEG


def _sc_gather(idx, table):
    mesh = plsc.VectorSubcoreMesh(core_axis_name="c", subcore_axis_name="s")

    @functools.partial(
        pl.kernel,
        mesh=mesh,
        out_type=jax.ShapeDtypeStruct((NUM_ROWS, DIM), jnp.float32),
        scratch_types=[
            pltpu.VMEM((GROUPS, SEG), jnp.int32),
            pltpu.VMEM((ROWS_PER_BATCH, DIM), jnp.float32),
            pltpu.SemaphoreType.DMA,
        ],
        compiler_params=pltpu.CompilerParams(use_tc_tiling_on_sc=False),
    )
    def k(idx_hbm, table_hbm, out_hbm, idx_v, rows_v, sem):
        wid = lax.axis_index("s") * NC + lax.axis_index("c")
        base = wid * PER_W
        pltpu.sync_copy(idx_hbm.at[wid], idx_v)

        def body(b, _):
            copies = []
            for j in range(K):
                copies.append(
                    pltpu.async_copy(
                        table_hbm.at[idx_v.at[b * K + j]],
                        rows_v.at[pl.ds(j * SEG, SEG)],
                        sem,
                    )
                )
            for c in copies:
                c.wait()
            pltpu.sync_copy(
                rows_v, out_hbm.at[pl.ds(base + b * ROWS_PER_BATCH, ROWS_PER_BATCH)]
            )
            return ()

        lax.fori_loop(0, NBATCH, body, ())

    return k(idx, table)


def kernel(token_ids, embeddings):
    idx = token_ids.reshape(NW, GROUPS, SEG).astype(jnp.int32)
    out = _sc_gather(idx, embeddings)
    return out.reshape(token_ids.shape + (DIM,))
